# baseline (speedup 1.0000x reference)
import functools

import jax
import jax.numpy as jnp
from jax import lax
from jax.experimental import pallas as pl
from jax.experimental.pallas import tpu as pltpu
from jax.experimental.pallas import tpu_sc as plsc

_B, _N, _DC, _DP = 16, 100000, 3, 8
_K = 16
_L = 16
_NPAD = 102400
_ROWS = 8
_RN = _NPAD // _ROWS
_HN = _RN // 2
_CN = 3200
_NCK = _HN // _CN
_G = 20
_GPTS = _G * _L
_NGROUP = _ROWS * _CN // _GPTS
_INF = float("inf")
_PAD_VAL = 1e30


def _merge16(bd, bi, d, i):
  sd, si = plsc.sort_key_val(d, i)
  rd = lax.rev(sd, (0,))
  ri = lax.rev(si, (0,))
  take_old = bd <= rd
  nd = jnp.where(take_old, bd, rd)
  ni = jnp.where(take_old, bi, ri)
  return plsc.sort_key_val(nd, ni)


def _sc_topk_idx(lgp, xs, ys):
  mesh = plsc.VectorSubcoreMesh(
      core_axis_name="c", subcore_axis_name="s", num_cores=2, num_subcores=16)

  @functools.partial(
      pl.kernel,
      out_type=jax.ShapeDtypeStruct((_B * _K,), jnp.int32),
      mesh=mesh,
      compiler_params=pltpu.CompilerParams(
          use_tc_tiling_on_sc=True, needs_layout_passes=False),
      scratch_types=[
          pltpu.VMEM((_ROWS, _CN), jnp.float32),
          pltpu.VMEM((_ROWS, _CN), jnp.float32),
          pltpu.VMEM((_ROWS, _CN), jnp.float32),
          pltpu.VMEM((_ROWS, _CN), jnp.float32),
          pltpu.VMEM((16,), jnp.float32),
          pltpu.VMEM((_K,), jnp.float32),
          pltpu.VMEM((_K,), jnp.int32),
          pltpu.VMEM((_K,), jnp.float32),
          pltpu.VMEM((_K,), jnp.int32),
          pltpu.VMEM_SHARED((16, _K), jnp.float32),
          pltpu.VMEM_SHARED((16, _K), jnp.int32),
          pltpu.SemaphoreType.DMA,
          pltpu.SemaphoreType.DMA,
          pltpu.SemaphoreType.DMA,
          pltpu.SemaphoreType.DMA,
          pltpu.SemaphoreType.DMA,
      ],
  )
  def scan_kernel(lgp_hbm, xs_hbm, ys_hbm, out_hbm,
                  xbuf0, xbuf1, ybuf0, ybuf1, lgp_v,
                  dv, iv, dv2, iv2, dsh, ish,
                  xsem0, xsem1, ysem0, ysem1, gsem):
    c = lax.axis_index("c")
    s = lax.axis_index("s")
    b = c * 8 + lax.rem(s, 8)
    h = s // 8

    iota = lax.iota(jnp.int32, _L)
    xbufs = (xbuf0, xbuf1)
    ybufs = (ybuf0, ybuf1)
    xsems = (xsem0, xsem1)
    ysems = (ysem0, ysem1)

    pltpu.sync_copy(lgp_hbm, lgp_v.at[pl.ds(0, 2)])
    lv = lgp_v[...]
    gx = lv[0]
    gy = lv[1]

    hbase = h * _HN
    cpx = pltpu.async_copy(
        xs_hbm.at[b, :, pl.ds(hbase, _CN)], xbuf0, xsem0)
    cpy = pltpu.async_copy(
        ys_hbm.at[b, :, pl.ds(hbase, _CN)], ybuf0, ysem0)

    best_d = jnp.full((_L,), _INF, jnp.float32)
    best_i = jnp.zeros((_L,), jnp.int32)
    thr = _INF

    for ck in range(_NCK):
      xb = xbufs[ck % 2]
      yb = ybufs[ck % 2]
      cpx.wait()
      cpy.wait()
      if ck + 1 < _NCK:
        cpx = pltpu.async_copy(
            xs_hbm.at[b, :, pl.ds(hbase + (ck + 1) * _CN, _CN)],
            xbufs[(ck + 1) % 2], xsems[(ck + 1) % 2])
        cpy = pltpu.async_copy(
            ys_hbm.at[b, :, pl.ds(hbase + (ck + 1) * _CN, _CN)],
            ybufs[(ck + 1) % 2], ysems[(ck + 1) % 2])

      cbase = hbase + ck * _CN

      def group_body(g, carry, xb=xb, yb=yb, cbase=cbase):
        del g
        bd, bi, th, r, cc = carry

        def dists(u):
          x = xb[r, pl.ds(cc + u * _L, _L)]
          y = yb[r, pl.ds(cc + u * _L, _L)]
          dx = x - gx
          dy = y - gy
          return dx * dx + dy * dy

        ds = [dists(u) for u in range(_G)]
        gmin = ds[0]
        for u in range(1, _G):
          gmin = jnp.minimum(gmin, ds[u])
        hit = jnp.min(gmin) < th
        pbase = r * _RN + cbase + cc

        def slow(bd, bi, th, pbase, *ds):
          mins = [jnp.min(du) for du in ds]
          for u in range(_G):
            du = ds[u]
            iu = pbase + u * _L + iota

            def do_merge(bd, bi, du=du, iu=iu):
              nd, ni = _merge16(bd, bi, du, iu)
              return nd, ni, jnp.max(nd)

            bd, bi, th = lax.cond(
                mins[u] < th,
                do_merge,
                lambda bd, bi, th=th: (bd, bi, th),
                bd, bi)
          return bd, bi, th

        bd, bi, th = lax.cond(
            hit, slow, lambda bd, bi, th, pbase, *ds: (bd, bi, th),
            bd, bi, th, pbase, *ds)

        cc2 = cc + _GPTS
        wrap = cc2 >= _CN
        r2 = jnp.where(wrap, r + 1, r)
        cc3 = jnp.where(wrap, 0, cc2)
        return bd, bi, th, r2, cc3

      if ck == 0:
        for u in range(_G):
          x0 = xb[0, pl.ds(u * _L, _L)]
          y0 = yb[0, pl.ds(u * _L, _L)]
          dx0 = x0 - gx
          dy0 = y0 - gy
          d0 = dx0 * dx0 + dy0 * dy0
          i0 = hbase + u * _L + iota
          best_d, best_i = _merge16(best_d, best_i, d0, i0)
        thr = jnp.max(best_d)
        g_lo, cc0 = 1, _GPTS
      else:
        g_lo, cc0 = 0, 0

      best_d, best_i, thr, _, _ = lax.fori_loop(
          g_lo, _NGROUP, group_body,
          (best_d, best_i, thr, jnp.int32(0), jnp.int32(cc0)))

    dv[...] = best_d
    iv[...] = best_i
    pltpu.sync_copy(dv, dsh.at[s])
    pltpu.sync_copy(iv, ish.at[s])
    plsc.subcore_barrier()

    @pl.when(h == 0)
    def _():
      pltpu.sync_copy(dsh.at[s + 8], dv2)
      pltpu.sync_copy(ish.at[s + 8], iv2)
      nd, ni = _merge16(best_d, best_i, dv2[...], iv2[...])
      iv[...] = ni
      pltpu.sync_copy(iv, out_hbm.at[pl.ds(b * _K, _K)])

  return scan_kernel(lgp, xs, ys)


def _mlp_kernel(lgp_ref, nc_ref, np_ref, w1_ref, b1_ref, w2_ref, b2_ref,
                w3_ref, b3_ref, out_ref):
  inv_k = jnp.float32(1.0 / _K)
  mean_xy = jnp.sum(nc_ref[...], axis=1) * inv_k
  mean_p = jnp.sum(np_ref[...], axis=1) * inv_k
  lgp = jnp.broadcast_to(lgp_ref[...], (_B, 2))
  x = jnp.concatenate([lgp, mean_xy, mean_p], axis=1)
  h = jnp.maximum(
      jnp.dot(x, w1_ref[...], preferred_element_type=jnp.float32)
      + b1_ref[...], 0.0)
  h = jnp.maximum(
      jnp.dot(h, w2_ref[...], preferred_element_type=jnp.float32)
      + b2_ref[...], 0.0)
  out_ref[...] = (
      jnp.dot(h, w3_ref[...], preferred_element_type=jnp.float32)
      + b3_ref[...])


def kernel(latent_grid_point, coords, params, W1, b1, W2, b2, W3, b3):
  pad = ((0, 0), (0, _NPAD - _N))
  xs = jnp.pad(coords[:, :, 0], pad,
               constant_values=_PAD_VAL).reshape(_B, _ROWS, _RN)
  ys = jnp.pad(coords[:, :, 1], pad,
               constant_values=_PAD_VAL).reshape(_B, _ROWS, _RN)
  idx = _sc_topk_idx(latent_grid_point, xs, ys).reshape(_B, _K)

  idxe = idx[:, :, None]
  ncoords = jnp.take_along_axis(coords[:, :, :2], idxe, axis=1)
  nparams = jnp.take_along_axis(params, idxe, axis=1)

  out = pl.pallas_call(
      _mlp_kernel,
      out_shape=jax.ShapeDtypeStruct((_B, 64), jnp.float32),
  )(latent_grid_point.reshape(1, 2), ncoords, nparams,
    W1, b1.reshape(1, 128), W2, b2.reshape(1, 128), W3, b3.reshape(1, 64))
  return out


# speedup vs baseline: 1.1081x; 1.1081x over previous
import functools

import jax
import jax.numpy as jnp
from jax import lax
from jax.experimental import pallas as pl
from jax.experimental.pallas import tpu as pltpu
from jax.experimental.pallas import tpu_sc as plsc

_B, _N, _DC, _DP = 16, 100000, 3, 8
_K = 16
_L = 16
_NPAD = 102400
_ROWS = 8
_RN = _NPAD // _ROWS
_HN = _RN // 2
_CN = 3200
_NCK = _HN // _CN
_G = 5
_GPTS = _G * _L
_NGROUP = _ROWS * _CN // _GPTS
_INF = float("inf")
_PAD_VAL = 1e30


def _merge16(bd, bi, d, i):
  sd, si = plsc.sort_key_val(d, i)
  rd = lax.rev(sd, (0,))
  ri = lax.rev(si, (0,))
  take_old = bd <= rd
  nd = jnp.where(take_old, bd, rd)
  ni = jnp.where(take_old, bi, ri)
  return plsc.sort_key_val(nd, ni)


def _sc_topk_idx(lgp, xs, ys):
  mesh = plsc.VectorSubcoreMesh(
      core_axis_name="c", subcore_axis_name="s", num_cores=2, num_subcores=16)

  @functools.partial(
      pl.kernel,
      out_type=jax.ShapeDtypeStruct((_B * _K,), jnp.int32),
      mesh=mesh,
      compiler_params=pltpu.CompilerParams(
          use_tc_tiling_on_sc=True, needs_layout_passes=False),
      scratch_types=[
          pltpu.VMEM((_ROWS, _CN), jnp.float32),
          pltpu.VMEM((_ROWS, _CN), jnp.float32),
          pltpu.VMEM((_ROWS, _CN), jnp.float32),
          pltpu.VMEM((_ROWS, _CN), jnp.float32),
          pltpu.VMEM((16,), jnp.float32),
          pltpu.VMEM((_K,), jnp.float32),
          pltpu.VMEM((_K,), jnp.int32),
          pltpu.VMEM((_K,), jnp.float32),
          pltpu.VMEM((_K,), jnp.int32),
          pltpu.VMEM_SHARED((16, _K), jnp.float32),
          pltpu.VMEM_SHARED((16, _K), jnp.int32),
          pltpu.SemaphoreType.DMA,
          pltpu.SemaphoreType.DMA,
          pltpu.SemaphoreType.DMA,
          pltpu.SemaphoreType.DMA,
          pltpu.SemaphoreType.DMA,
      ],
  )
  def scan_kernel(lgp_hbm, xs_hbm, ys_hbm, out_hbm,
                  xbuf0, xbuf1, ybuf0, ybuf1, lgp_v,
                  dv, iv, dv2, iv2, dsh, ish,
                  xsem0, xsem1, ysem0, ysem1, gsem):
    c = lax.axis_index("c")
    s = lax.axis_index("s")
    b = c * 8 + lax.rem(s, 8)
    h = s // 8

    iota = lax.iota(jnp.int32, _L)
    xbufs = (xbuf0, xbuf1)
    ybufs = (ybuf0, ybuf1)
    xsems = (xsem0, xsem1)
    ysems = (ysem0, ysem1)

    pltpu.sync_copy(lgp_hbm, lgp_v.at[pl.ds(0, 2)])
    lv = lgp_v[...]
    gx = lv[0]
    gy = lv[1]

    hbase = h * _HN
    cpx = pltpu.async_copy(
        xs_hbm.at[b, :, pl.ds(hbase, _CN)], xbuf0, xsem0)
    cpy = pltpu.async_copy(
        ys_hbm.at[b, :, pl.ds(hbase, _CN)], ybuf0, ysem0)

    best_d = jnp.full((_L,), _INF, jnp.float32)
    best_i = jnp.zeros((_L,), jnp.int32)
    thr = _INF

    for ck in range(_NCK):
      xb = xbufs[ck % 2]
      yb = ybufs[ck % 2]
      cpx.wait()
      cpy.wait()
      if ck + 1 < _NCK:
        cpx = pltpu.async_copy(
            xs_hbm.at[b, :, pl.ds(hbase + (ck + 1) * _CN, _CN)],
            xbufs[(ck + 1) % 2], xsems[(ck + 1) % 2])
        cpy = pltpu.async_copy(
            ys_hbm.at[b, :, pl.ds(hbase + (ck + 1) * _CN, _CN)],
            ybufs[(ck + 1) % 2], ysems[(ck + 1) % 2])

      cbase = hbase + ck * _CN

      def group_body(g, carry, xb=xb, yb=yb, cbase=cbase):
        del g
        bd, bi, th, r, cc = carry

        def dists(u):
          x = xb[r, pl.ds(cc + u * _L, _L)]
          y = yb[r, pl.ds(cc + u * _L, _L)]
          dx = x - gx
          dy = y - gy
          return dx * dx + dy * dy

        ds = [dists(u) for u in range(_G)]
        gmin = ds[0]
        for u in range(1, _G):
          gmin = jnp.minimum(gmin, ds[u])
        hit = jnp.min(gmin) < th
        pbase = r * _RN + cbase + cc

        def slow(bd, bi, th, pbase, *ds):
          mins = [jnp.min(du) for du in ds]
          for u in range(_G):
            du = ds[u]
            iu = pbase + u * _L + iota

            def do_merge(bd, bi, du=du, iu=iu):
              nd, ni = _merge16(bd, bi, du, iu)
              return nd, ni, jnp.max(nd)

            bd, bi, th = lax.cond(
                mins[u] < th,
                do_merge,
                lambda bd, bi, th=th: (bd, bi, th),
                bd, bi)
          return bd, bi, th

        bd, bi, th = lax.cond(
            hit, slow, lambda bd, bi, th, pbase, *ds: (bd, bi, th),
            bd, bi, th, pbase, *ds)

        cc2 = cc + _GPTS
        wrap = cc2 >= _CN
        r2 = jnp.where(wrap, r + 1, r)
        cc3 = jnp.where(wrap, 0, cc2)
        return bd, bi, th, r2, cc3

      if ck == 0:
        for u in range(_G):
          x0 = xb[0, pl.ds(u * _L, _L)]
          y0 = yb[0, pl.ds(u * _L, _L)]
          dx0 = x0 - gx
          dy0 = y0 - gy
          d0 = dx0 * dx0 + dy0 * dy0
          i0 = hbase + u * _L + iota
          best_d, best_i = _merge16(best_d, best_i, d0, i0)
        thr = jnp.max(best_d)
        g_lo, cc0 = 1, _GPTS
      else:
        g_lo, cc0 = 0, 0

      best_d, best_i, thr, _, _ = lax.fori_loop(
          g_lo, _NGROUP, group_body,
          (best_d, best_i, thr, jnp.int32(0), jnp.int32(cc0)))

    dv[...] = best_d
    iv[...] = best_i
    pltpu.sync_copy(dv, dsh.at[s])
    pltpu.sync_copy(iv, ish.at[s])
    plsc.subcore_barrier()

    @pl.when(h == 0)
    def _():
      pltpu.sync_copy(dsh.at[s + 8], dv2)
      pltpu.sync_copy(ish.at[s + 8], iv2)
      nd, ni = _merge16(best_d, best_i, dv2[...], iv2[...])
      iv[...] = ni
      pltpu.sync_copy(iv, out_hbm.at[pl.ds(b * _K, _K)])

  return scan_kernel(lgp, xs, ys)


def _mlp_kernel(lgp_ref, nc_ref, np_ref, w1_ref, b1_ref, w2_ref, b2_ref,
                w3_ref, b3_ref, out_ref):
  inv_k = jnp.float32(1.0 / _K)
  mean_xy = jnp.sum(nc_ref[...], axis=1) * inv_k
  mean_p = jnp.sum(np_ref[...], axis=1) * inv_k
  lgp = jnp.broadcast_to(lgp_ref[...], (_B, 2))
  x = jnp.concatenate([lgp, mean_xy, mean_p], axis=1)
  h = jnp.maximum(
      jnp.dot(x, w1_ref[...], preferred_element_type=jnp.float32)
      + b1_ref[...], 0.0)
  h = jnp.maximum(
      jnp.dot(h, w2_ref[...], preferred_element_type=jnp.float32)
      + b2_ref[...], 0.0)
  out_ref[...] = (
      jnp.dot(h, w3_ref[...], preferred_element_type=jnp.float32)
      + b3_ref[...])


def kernel(latent_grid_point, coords, params, W1, b1, W2, b2, W3, b3):
  pad = ((0, 0), (0, _NPAD - _N))
  xs = jnp.pad(coords[:, :, 0], pad,
               constant_values=_PAD_VAL).reshape(_B, _ROWS, _RN)
  ys = jnp.pad(coords[:, :, 1], pad,
               constant_values=_PAD_VAL).reshape(_B, _ROWS, _RN)
  idx = _sc_topk_idx(latent_grid_point, xs, ys).reshape(_B, _K)

  idxe = idx[:, :, None]
  ncoords = jnp.take_along_axis(coords[:, :, :2], idxe, axis=1)
  nparams = jnp.take_along_axis(params, idxe, axis=1)

  out = pl.pallas_call(
      _mlp_kernel,
      out_shape=jax.ShapeDtypeStruct((_B, 64), jnp.float32),
  )(latent_grid_point.reshape(1, 2), ncoords, nparams,
    W1, b1.reshape(1, 128), W2, b2.reshape(1, 128), W3, b3.reshape(1, 64))
  return out


# speedup vs baseline: 1.1122x; 1.0037x over previous
import functools

import jax
import jax.numpy as jnp
from jax import lax
from jax.experimental import pallas as pl
from jax.experimental.pallas import tpu as pltpu
from jax.experimental.pallas import tpu_sc as plsc

_B, _N, _DC, _DP = 16, 100000, 3, 8
_K = 16
_L = 16
_NPAD = 102400
_ROWS = 8
_RN = _NPAD // _ROWS
_HN = _RN // 2
_CN = 3200
_NCK = _HN // _CN
_G = 8
_GPTS = _G * _L
_NGROUP = _ROWS * _CN // _GPTS
_INF = float("inf")
_PAD_VAL = 1e30


def _merge16(bd, bi, d, i):
  sd, si = plsc.sort_key_val(d, i)
  rd = lax.rev(sd, (0,))
  ri = lax.rev(si, (0,))
  take_old = bd <= rd
  nd = jnp.where(take_old, bd, rd)
  ni = jnp.where(take_old, bi, ri)
  return plsc.sort_key_val(nd, ni)


def _sc_topk_idx(lgp, xs, ys):
  mesh = plsc.VectorSubcoreMesh(
      core_axis_name="c", subcore_axis_name="s", num_cores=2, num_subcores=16)

  @functools.partial(
      pl.kernel,
      out_type=jax.ShapeDtypeStruct((_B * _K,), jnp.int32),
      mesh=mesh,
      compiler_params=pltpu.CompilerParams(
          use_tc_tiling_on_sc=True, needs_layout_passes=False),
      scratch_types=[
          pltpu.VMEM((_ROWS, _CN), jnp.float32),
          pltpu.VMEM((_ROWS, _CN), jnp.float32),
          pltpu.VMEM((_ROWS, _CN), jnp.float32),
          pltpu.VMEM((_ROWS, _CN), jnp.float32),
          pltpu.VMEM((16,), jnp.float32),
          pltpu.VMEM((_K,), jnp.float32),
          pltpu.VMEM((_K,), jnp.int32),
          pltpu.VMEM((_K,), jnp.float32),
          pltpu.VMEM((_K,), jnp.int32),
          pltpu.VMEM_SHARED((16, _K), jnp.float32),
          pltpu.VMEM_SHARED((16, _K), jnp.int32),
          pltpu.SemaphoreType.DMA,
          pltpu.SemaphoreType.DMA,
          pltpu.SemaphoreType.DMA,
          pltpu.SemaphoreType.DMA,
          pltpu.SemaphoreType.DMA,
      ],
  )
  def scan_kernel(lgp_hbm, xs_hbm, ys_hbm, out_hbm,
                  xbuf0, xbuf1, ybuf0, ybuf1, lgp_v,
                  dv, iv, dv2, iv2, dsh, ish,
                  xsem0, xsem1, ysem0, ysem1, gsem):
    c = lax.axis_index("c")
    s = lax.axis_index("s")
    b = c * 8 + lax.rem(s, 8)
    h = s // 8

    iota = lax.iota(jnp.int32, _L)
    xbufs = (xbuf0, xbuf1)
    ybufs = (ybuf0, ybuf1)
    xsems = (xsem0, xsem1)
    ysems = (ysem0, ysem1)

    pltpu.sync_copy(lgp_hbm, lgp_v.at[pl.ds(0, 2)])
    lv = lgp_v[...]
    gx = lv[0]
    gy = lv[1]

    hbase = h * _HN
    cpx = pltpu.async_copy(
        xs_hbm.at[b, :, pl.ds(hbase, _CN)], xbuf0, xsem0)
    cpy = pltpu.async_copy(
        ys_hbm.at[b, :, pl.ds(hbase, _CN)], ybuf0, ysem0)

    best_d = jnp.full((_L,), _INF, jnp.float32)
    best_i = jnp.zeros((_L,), jnp.int32)
    thr = _INF

    for ck in range(_NCK):
      xb = xbufs[ck % 2]
      yb = ybufs[ck % 2]
      cpx.wait()
      cpy.wait()
      if ck + 1 < _NCK:
        cpx = pltpu.async_copy(
            xs_hbm.at[b, :, pl.ds(hbase + (ck + 1) * _CN, _CN)],
            xbufs[(ck + 1) % 2], xsems[(ck + 1) % 2])
        cpy = pltpu.async_copy(
            ys_hbm.at[b, :, pl.ds(hbase + (ck + 1) * _CN, _CN)],
            ybufs[(ck + 1) % 2], ysems[(ck + 1) % 2])

      cbase = hbase + ck * _CN

      def group_body(g, carry, xb=xb, yb=yb, cbase=cbase):
        del g
        bd, bi, th, r, cc = carry

        def dists(u):
          x = xb[r, pl.ds(cc + u * _L, _L)]
          y = yb[r, pl.ds(cc + u * _L, _L)]
          dx = x - gx
          dy = y - gy
          return dx * dx + dy * dy

        ds = [dists(u) for u in range(_G)]
        gmin = ds[0]
        for u in range(1, _G):
          gmin = jnp.minimum(gmin, ds[u])
        hit = jnp.min(gmin) < th
        pbase = r * _RN + cbase + cc

        def slow(bd, bi, th, pbase, *ds):
          mins = [jnp.min(du) for du in ds]
          for u in range(_G):
            du = ds[u]
            iu = pbase + u * _L + iota

            def do_merge(bd, bi, du=du, iu=iu):
              nd, ni = _merge16(bd, bi, du, iu)
              return nd, ni, jnp.max(nd)

            bd, bi, th = lax.cond(
                mins[u] < th,
                do_merge,
                lambda bd, bi, th=th: (bd, bi, th),
                bd, bi)
          return bd, bi, th

        bd, bi, th = lax.cond(
            hit, slow, lambda bd, bi, th, pbase, *ds: (bd, bi, th),
            bd, bi, th, pbase, *ds)

        cc2 = cc + _GPTS
        wrap = cc2 >= _CN
        r2 = jnp.where(wrap, r + 1, r)
        cc3 = jnp.where(wrap, 0, cc2)
        return bd, bi, th, r2, cc3

      if ck == 0:
        for u in range(_G):
          x0 = xb[0, pl.ds(u * _L, _L)]
          y0 = yb[0, pl.ds(u * _L, _L)]
          dx0 = x0 - gx
          dy0 = y0 - gy
          d0 = dx0 * dx0 + dy0 * dy0
          i0 = hbase + u * _L + iota
          best_d, best_i = _merge16(best_d, best_i, d0, i0)
        thr = jnp.max(best_d)
        g_lo, cc0 = 1, _GPTS
      else:
        g_lo, cc0 = 0, 0

      best_d, best_i, thr, _, _ = lax.fori_loop(
          g_lo, _NGROUP, group_body,
          (best_d, best_i, thr, jnp.int32(0), jnp.int32(cc0)))

    dv[...] = best_d
    iv[...] = best_i
    pltpu.sync_copy(dv, dsh.at[s])
    pltpu.sync_copy(iv, ish.at[s])
    plsc.subcore_barrier()

    @pl.when(h == 0)
    def _():
      pltpu.sync_copy(dsh.at[s + 8], dv2)
      pltpu.sync_copy(ish.at[s + 8], iv2)
      nd, ni = _merge16(best_d, best_i, dv2[...], iv2[...])
      iv[...] = ni
      pltpu.sync_copy(iv, out_hbm.at[pl.ds(b * _K, _K)])

  return scan_kernel(lgp, xs, ys)


def _mlp_kernel(lgp_ref, nc_ref, np_ref, w1_ref, b1_ref, w2_ref, b2_ref,
                w3_ref, b3_ref, out_ref):
  inv_k = jnp.float32(1.0 / _K)
  mean_xy = jnp.sum(nc_ref[...], axis=1) * inv_k
  mean_p = jnp.sum(np_ref[...], axis=1) * inv_k
  lgp = jnp.broadcast_to(lgp_ref[...], (_B, 2))
  x = jnp.concatenate([lgp, mean_xy, mean_p], axis=1)
  h = jnp.maximum(
      jnp.dot(x, w1_ref[...], preferred_element_type=jnp.float32)
      + b1_ref[...], 0.0)
  h = jnp.maximum(
      jnp.dot(h, w2_ref[...], preferred_element_type=jnp.float32)
      + b2_ref[...], 0.0)
  out_ref[...] = (
      jnp.dot(h, w3_ref[...], preferred_element_type=jnp.float32)
      + b3_ref[...])


def kernel(latent_grid_point, coords, params, W1, b1, W2, b2, W3, b3):
  pad = ((0, 0), (0, _NPAD - _N))
  xs = jnp.pad(coords[:, :, 0], pad,
               constant_values=_PAD_VAL).reshape(_B, _ROWS, _RN)
  ys = jnp.pad(coords[:, :, 1], pad,
               constant_values=_PAD_VAL).reshape(_B, _ROWS, _RN)
  idx = _sc_topk_idx(latent_grid_point, xs, ys).reshape(_B, _K)

  idxe = idx[:, :, None]
  ncoords = jnp.take_along_axis(coords[:, :, :2], idxe, axis=1)
  nparams = jnp.take_along_axis(params, idxe, axis=1)

  out = pl.pallas_call(
      _mlp_kernel,
      out_shape=jax.ShapeDtypeStruct((_B, 64), jnp.float32),
  )(latent_grid_point.reshape(1, 2), ncoords, nparams,
    W1, b1.reshape(1, 128), W2, b2.reshape(1, 128), W3, b3.reshape(1, 64))
  return out
